# baseline (speedup 1.0000x reference)
import jax
import jax.numpy as jnp
from jax.experimental import pallas as pl
from jax.experimental.pallas import tpu as pltpu

B = 4096
L = 200
D = 32
H1 = 64
H2 = 32
WINDOWS = (10, 20, 50, 120)
CUMSUM = (0, 10, 30, 80)
BB = 64
P = 4
G = L // P


GP = 56


def _din_block(q_ref, qt_ref, seq_ref, len_ref, w1a_ref, w1b_ref, w1c_ref,
               b1_ref, w2_ref, b2_ref, wlmat_ref, bl_ref, alpha_ref,
               tpos_ref, fold_ref, out_ref):
    f32 = jnp.float32
    q = q_ref[:]
    qt = qt_ref[:]
    seq = seq_ref[:]
    seqp = jnp.pad(seq, ((0, 0), (0, GP - G), (0, 0)))
    qsp = seqp * qt[:, None, :]
    h = (jnp.dot(seqp.reshape(BB * GP, P * D), w1a_ref[:],
                 preferred_element_type=f32)
         + jnp.dot(qsp.reshape(BB * GP, P * D), w1b_ref[:],
                   preferred_element_type=f32))
    qc = jnp.dot(q, w1c_ref[:], preferred_element_type=f32) + b1_ref[:]
    qct = jnp.concatenate([qc] * P, axis=1)
    h = jnp.maximum(h.reshape(BB, GP, P * H1) + qct[:, None, :], 0.0)
    h2 = jnp.maximum(jnp.dot(h.reshape(BB * GP, P * H1), w2_ref[:],
                             preferred_element_type=f32) + b2_ref[:], 0.0)
    araw = jnp.dot(h2, wlmat_ref[:], preferred_element_type=f32) + bl_ref[0, 0]
    a3 = araw.reshape(BB, GP, P * D)
    seq = seqp
    slen = len_ref[:]
    tpos = tpos_ref[:]
    av = jnp.where(tpos < slen[:, :, None], a3, 0.0)
    alpha = alpha_ref[0, 0]
    w3 = jnp.where(av > 0, av, alpha * av)
    weighted = w3 * seq

    lane = jax.lax.broadcasted_iota(jnp.int32, (BB, P * D), 1)
    first_half = lane < 2 * D
    g2 = weighted[:, 2, :]
    g7 = weighted[:, 7, :]
    zero = jnp.zeros_like(g2)
    win0 = (jnp.sum(weighted[:, 0:2, :], axis=1)
            + jnp.where(first_half, g2, zero))
    win1 = (jnp.sum(weighted[:, 3:7, :], axis=1)
            + jnp.where(first_half, zero, g2)
            + jnp.where(first_half, g7, zero))
    win2 = (jnp.sum(weighted[:, 8:20, :], axis=1)
            + jnp.where(first_half, zero, g7))
    win3 = jnp.sum(weighted[:, 20:50, :], axis=1)

    parts = []
    for wsum, st, wn in zip((win0, win1, win2, win3), CUMSUM, WINDOWS):
        denom = jnp.maximum(jnp.minimum(slen - st, wn), 1).astype(f32)
        parts.append(wsum / denom)
    wcat = jnp.concatenate(parts, axis=-1)
    folded = jnp.dot(wcat, fold_ref[:], preferred_element_type=f32)
    out_ref[:] = jnp.concatenate([folded, q], axis=-1)


def kernel(query, sequence, sequence_length, W1, b1, W2, b2, Wl, bl, alpha):
    slen = sequence_length.astype(jnp.int32).reshape(B, 1)
    eye = jnp.eye(P, dtype=jnp.float32)
    w1a = jnp.kron(eye, W1[:D])
    w1b = jnp.kron(eye, W1[D:2 * D])
    w1c = W1[2 * D:]
    b1r = b1.reshape(1, H1)
    b2r = jnp.tile(b2.reshape(1, H2), (1, P))
    w2bd = jnp.kron(eye, W2)
    wlmat = jnp.kron(eye, jnp.tile(Wl, (1, D)))
    blr = bl.reshape(1, 1)
    alphar = alpha.reshape(1, 1)
    seqp = sequence.reshape(B, G, P * D)
    qtile = jnp.tile(query, (1, P))
    tpos = (P * jnp.arange(GP, dtype=jnp.int32)[None, :, None]
            + (jnp.arange(P * D, dtype=jnp.int32) // D)[None, None, :])
    foldm = jnp.kron(jnp.eye(4, dtype=jnp.float32),
                     jnp.tile(jnp.eye(D, dtype=jnp.float32), (P, 1)))

    grid = (B // BB,)
    full = lambda *s: pl.BlockSpec(s, lambda i: (0,) * len(s))
    out = pl.pallas_call(
        _din_block,
        grid=grid,
        in_specs=[
            pl.BlockSpec((BB, D), lambda i: (i, 0)),
            pl.BlockSpec((BB, P * D), lambda i: (i, 0)),
            pl.BlockSpec((BB, G, P * D), lambda i: (i, 0, 0)),
            pl.BlockSpec((BB, 1), lambda i: (i, 0)),
            full(P * D, P * H1),
            full(P * D, P * H1),
            full(D, H1),
            full(1, H1),
            full(P * H1, P * H2),
            full(1, P * H2),
            full(P * H2, P * D),
            full(1, 1),
            full(1, 1),
            full(1, GP, P * D),
            full(4 * P * D, 4 * D),
        ],
        out_specs=pl.BlockSpec((BB, 5 * D), lambda i: (i, 0)),
        out_shape=jax.ShapeDtypeStruct((B, 5 * D), jnp.float32),
        compiler_params=pltpu.CompilerParams(
            dimension_semantics=("parallel",),
        ),
    )(query, qtile, seqp, slen, w1a, w1b, w1c, b1r, w2bd, b2r, wlmat, blr,
      alphar, tpos, foldm)
    return out


# speedup vs baseline: 1.2513x; 1.2513x over previous
import jax
import jax.numpy as jnp
from jax.experimental import pallas as pl
from jax.experimental.pallas import tpu as pltpu

B = 4096
L = 200
D = 32
H1 = 64
H2 = 32
WINDOWS = (10, 20, 50, 120)
CUMSUM = (0, 10, 30, 80)
BB = 256
P = 4
G = L // P


GP = 56


def _din_block(q_ref, qt_ref, seq_ref, len_ref, w1_ref, w1c_ref,
               b1_ref, w2_ref, b2_ref, wlmat_ref, bl_ref, alpha_ref,
               tpos_ref, fold_ref, out_ref):
    f32 = jnp.float32
    bf16 = jnp.bfloat16
    q = q_ref[:]
    qt = qt_ref[:]
    seq = seq_ref[:]
    seqb50 = seq.astype(bf16)
    seqp = jnp.pad(seqb50, ((0, 0), (0, GP - G), (0, 0)))
    xcat = jnp.concatenate([seqp, seqp * qt[:, None, :]], axis=2)
    h = jnp.dot(xcat.reshape(BB * GP, 2 * P * D), w1_ref[:],
                preferred_element_type=f32).astype(bf16)
    qc = jnp.dot(qt[:, :D], w1c_ref[:],
                 preferred_element_type=f32) + b1_ref[:]
    qct = jnp.concatenate([qc.astype(bf16)] * P, axis=1)
    h = jnp.maximum(h.reshape(BB, GP, P * H1) + qct[:, None, :], bf16(0.0))
    h2 = (jnp.dot(h.reshape(BB * GP, P * H1), w2_ref[:],
                  preferred_element_type=f32) + b2_ref[:]).astype(bf16)
    h2 = jnp.maximum(h2, bf16(0.0))
    araw = jnp.dot(h2, wlmat_ref[:], preferred_element_type=f32) + bl_ref[0, 0]
    a3 = araw.reshape(BB, GP, P * D)[:, :G, :]
    slen = len_ref[:]
    tpos = tpos_ref[:]
    av = jnp.where(tpos < slen[:, :, None], a3, 0.0)
    alpha = alpha_ref[0, 0]
    w3 = jnp.where(av > 0, av, alpha * av)
    weighted = w3 * seq

    lane = jax.lax.broadcasted_iota(jnp.int32, (BB, P * D), 1)
    first_half = lane < 2 * D
    g2 = weighted[:, 2, :]
    g7 = weighted[:, 7, :]
    zero = jnp.zeros_like(g2)
    win0 = (jnp.sum(weighted[:, 0:2, :], axis=1)
            + jnp.where(first_half, g2, zero))
    win1 = (jnp.sum(weighted[:, 3:7, :], axis=1)
            + jnp.where(first_half, zero, g2)
            + jnp.where(first_half, g7, zero))
    win2 = (jnp.sum(weighted[:, 8:20, :], axis=1)
            + jnp.where(first_half, zero, g7))
    win3 = jnp.sum(weighted[:, 20:50, :], axis=1)

    parts = []
    for wsum, st, wn in zip((win0, win1, win2, win3), CUMSUM, WINDOWS):
        denom = jnp.maximum(jnp.minimum(slen - st, wn), 1).astype(f32)
        parts.append(wsum * (1.0 / denom))
    wcat = jnp.concatenate(parts, axis=-1)
    folded = jnp.dot(wcat, fold_ref[:], preferred_element_type=f32)
    out_ref[:] = jnp.concatenate([folded, q], axis=-1)


def kernel(query, sequence, sequence_length, W1, b1, W2, b2, Wl, bl, alpha):
    bf16 = jnp.bfloat16
    slen = sequence_length.astype(jnp.int32).reshape(B, 1)
    eye = jnp.eye(P, dtype=jnp.float32)
    w1cat = jnp.concatenate([jnp.kron(eye, W1[:D]),
                             jnp.kron(eye, W1[D:2 * D])],
                            axis=0).astype(bf16)
    w1c = W1[2 * D:].astype(bf16)
    b1r = b1.reshape(1, H1)
    b2r = jnp.tile(b2.reshape(1, H2), (1, P))
    w2bd = jnp.kron(eye, W2).astype(bf16)
    wlmat = jnp.kron(eye, jnp.tile(Wl, (1, D))).astype(bf16)
    blr = bl.reshape(1, 1)
    alphar = alpha.reshape(1, 1)
    seqp = sequence.reshape(B, G, P * D)
    qtile = jnp.tile(query, (1, P)).astype(bf16)
    tpos = (P * jnp.arange(G, dtype=jnp.int32)[None, :, None]
            + (jnp.arange(P * D, dtype=jnp.int32) // D)[None, None, :])
    foldm = jnp.kron(jnp.eye(4, dtype=jnp.float32),
                     jnp.tile(jnp.eye(D, dtype=jnp.float32), (P, 1)))

    grid = (B // BB,)
    full = lambda *s: pl.BlockSpec(s, lambda i: (0,) * len(s))
    out = pl.pallas_call(
        _din_block,
        grid=grid,
        in_specs=[
            pl.BlockSpec((BB, D), lambda i: (i, 0)),
            pl.BlockSpec((BB, P * D), lambda i: (i, 0)),
            pl.BlockSpec((BB, G, P * D), lambda i: (i, 0, 0)),
            pl.BlockSpec((BB, 1), lambda i: (i, 0)),
            full(2 * P * D, P * H1),
            full(D, H1),
            full(1, H1),
            full(P * H1, P * H2),
            full(1, P * H2),
            full(P * H2, P * D),
            full(1, 1),
            full(1, 1),
            full(1, G, P * D),
            full(4 * P * D, 4 * D),
        ],
        out_specs=pl.BlockSpec((BB, 5 * D), lambda i: (i, 0)),
        out_shape=jax.ShapeDtypeStruct((B, 5 * D), jnp.float32),
        compiler_params=pltpu.CompilerParams(
            dimension_semantics=("parallel",),
        ),
    )(query, qtile, seqp, slen, w1cat, w1c, b1r, w2bd, b2r, wlmat, blr,
      alphar, tpos, foldm)
    return out


# speedup vs baseline: 1.2545x; 1.0026x over previous
import jax
import jax.numpy as jnp
from jax.experimental import pallas as pl
from jax.experimental.pallas import tpu as pltpu

B = 4096
L = 200
D = 32
H1 = 64
H2 = 32
WINDOWS = (10, 20, 50, 120)
CUMSUM = (0, 10, 30, 80)
BB = 256
P = 4
G = L // P


GP = 56


def _din_block(q_ref, qt_ref, seq_ref, len_ref, w1_ref, w1c_ref,
               b1_ref, w2_ref, b2_ref, wlmat_ref, bl_ref, alpha_ref,
               tpos_ref, fold_ref, out_ref):
    f32 = jnp.float32
    bf16 = jnp.bfloat16
    q = q_ref[:]
    qt = qt_ref[:]
    seq = seq_ref[:]
    seqb50 = seq.astype(bf16)
    seqp = jnp.pad(seqb50, ((0, 0), (0, GP - G), (0, 0)))
    xcat = jnp.concatenate([seqp, seqp * qt[:, None, :]], axis=2)
    h = jnp.dot(xcat.reshape(BB * GP, 2 * P * D), w1_ref[:],
                preferred_element_type=f32).astype(bf16)
    qct = (jnp.dot(qt[:, :D], w1c_ref[:], preferred_element_type=f32)
           + b1_ref[:]).astype(bf16)
    h = jnp.maximum(h.reshape(BB, GP, P * H1) + qct[:, None, :], bf16(0.0))
    h2 = (jnp.dot(h.reshape(BB * GP, P * H1), w2_ref[:],
                  preferred_element_type=f32) + b2_ref[:]).astype(bf16)
    h2 = jnp.maximum(h2, bf16(0.0))
    araw = jnp.dot(h2, wlmat_ref[:], preferred_element_type=f32) + bl_ref[0, 0]
    a3 = araw.reshape(BB, GP, P * D)[:, :G, :]
    slen = len_ref[:]
    tpos = tpos_ref[:]
    av = jnp.where(tpos < slen[:, :, None], a3, 0.0)
    alpha = alpha_ref[0, 0]
    w3 = jnp.where(av > 0, av, alpha * av)
    weighted = w3 * seq

    lane = jax.lax.broadcasted_iota(jnp.int32, (BB, P * D), 1)
    first_half = lane < 2 * D
    g2 = weighted[:, 2, :]
    g7 = weighted[:, 7, :]
    zero = jnp.zeros_like(g2)
    win0 = (jnp.sum(weighted[:, 0:2, :], axis=1)
            + jnp.where(first_half, g2, zero))
    win1 = (jnp.sum(weighted[:, 3:7, :], axis=1)
            + jnp.where(first_half, zero, g2)
            + jnp.where(first_half, g7, zero))
    win2 = (jnp.sum(weighted[:, 8:20, :], axis=1)
            + jnp.where(first_half, zero, g7))
    win3 = jnp.sum(weighted[:, 20:50, :], axis=1)

    parts = []
    for wsum, st, wn in zip((win0, win1, win2, win3), CUMSUM, WINDOWS):
        folded = jnp.dot(wsum, fold_ref[:], preferred_element_type=f32)
        denom = jnp.maximum(jnp.minimum(slen - st, wn), 1).astype(f32)
        parts.append(folded * (1.0 / denom))
    parts.append(q)
    out_ref[:] = jnp.concatenate(parts, axis=-1)


def kernel(query, sequence, sequence_length, W1, b1, W2, b2, Wl, bl, alpha):
    bf16 = jnp.bfloat16
    slen = sequence_length.astype(jnp.int32).reshape(B, 1)
    eye = jnp.eye(P, dtype=jnp.float32)
    w1cat = jnp.concatenate([jnp.kron(eye, W1[:D]),
                             jnp.kron(eye, W1[D:2 * D])],
                            axis=0).astype(bf16)
    w1c = jnp.tile(W1[2 * D:], (1, P)).astype(bf16)
    b1r = jnp.tile(b1.reshape(1, H1), (1, P))
    b2r = jnp.tile(b2.reshape(1, H2), (1, P))
    w2bd = jnp.kron(eye, W2).astype(bf16)
    wlmat = jnp.kron(eye, jnp.tile(Wl, (1, D))).astype(bf16)
    blr = bl.reshape(1, 1)
    alphar = alpha.reshape(1, 1)
    seqp = sequence.reshape(B, G, P * D)
    qtile = jnp.tile(query, (1, P)).astype(bf16)
    tpos = (P * jnp.arange(G, dtype=jnp.int32)[None, :, None]
            + (jnp.arange(P * D, dtype=jnp.int32) // D)[None, None, :])
    foldm = jnp.tile(jnp.eye(D, dtype=jnp.float32), (P, 1))

    grid = (B // BB,)
    full = lambda *s: pl.BlockSpec(s, lambda i: (0,) * len(s))
    out = pl.pallas_call(
        _din_block,
        grid=grid,
        in_specs=[
            pl.BlockSpec((BB, D), lambda i: (i, 0)),
            pl.BlockSpec((BB, P * D), lambda i: (i, 0)),
            pl.BlockSpec((BB, G, P * D), lambda i: (i, 0, 0)),
            pl.BlockSpec((BB, 1), lambda i: (i, 0)),
            full(2 * P * D, P * H1),
            full(D, P * H1),
            full(1, P * H1),
            full(P * H1, P * H2),
            full(1, P * H2),
            full(P * H2, P * D),
            full(1, 1),
            full(1, 1),
            full(1, G, P * D),
            full(P * D, D),
        ],
        out_specs=pl.BlockSpec((BB, 5 * D), lambda i: (i, 0)),
        out_shape=jax.ShapeDtypeStruct((B, 5 * D), jnp.float32),
        compiler_params=pltpu.CompilerParams(
            dimension_semantics=("parallel",),
        ),
    )(query, qtile, seqp, slen, w1cat, w1c, b1r, w2bd, b2r, wlmat, blr,
      alphar, tpos, foldm)
    return out


# speedup vs baseline: 1.2568x; 1.0018x over previous
import jax
import jax.numpy as jnp
from jax.experimental import pallas as pl
from jax.experimental.pallas import tpu as pltpu

B = 4096
L = 200
D = 32
H1 = 64
H2 = 32
WINDOWS = (10, 20, 50, 120)
CUMSUM = (0, 10, 30, 80)
BB = 512
P = 4
G = L // P


GP = 56


def _din_block(q_ref, qt_ref, seq_ref, len_ref, w1_ref, w1c_ref,
               b1_ref, w2_ref, b2_ref, wlmat_ref, bl_ref, alpha_ref,
               tpos_ref, fold_ref, out_ref):
    f32 = jnp.float32
    bf16 = jnp.bfloat16
    q = q_ref[:]
    qt = qt_ref[:]
    seq = seq_ref[:]
    seqb50 = seq.astype(bf16)
    seqp = jnp.pad(seqb50, ((0, 0), (0, GP - G), (0, 0)))
    xcat = jnp.concatenate([seqp, seqp * qt[:, None, :]], axis=2)
    h = jnp.dot(xcat.reshape(BB * GP, 2 * P * D), w1_ref[:],
                preferred_element_type=f32).astype(bf16)
    qct = (jnp.dot(qt[:, :D], w1c_ref[:], preferred_element_type=f32)
           + b1_ref[:]).astype(bf16)
    h = jnp.maximum(h.reshape(BB, GP, P * H1) + qct[:, None, :], bf16(0.0))
    h2 = (jnp.dot(h.reshape(BB * GP, P * H1), w2_ref[:],
                  preferred_element_type=f32) + b2_ref[:]).astype(bf16)
    h2 = jnp.maximum(h2, bf16(0.0))
    araw = jnp.dot(h2, wlmat_ref[:], preferred_element_type=f32) + bl_ref[0, 0]
    a3 = araw.reshape(BB, GP, P * D)[:, :G, :]
    slen = len_ref[:]
    tpos = tpos_ref[:]
    av = jnp.where(tpos < slen[:, :, None], a3, 0.0)
    alpha = alpha_ref[0, 0]
    w3 = jnp.where(av > 0, av, alpha * av)
    weighted = w3 * seq

    lane = jax.lax.broadcasted_iota(jnp.int32, (BB, P * D), 1)
    first_half = lane < 2 * D
    g2 = weighted[:, 2, :]
    g7 = weighted[:, 7, :]
    zero = jnp.zeros_like(g2)
    win0 = (jnp.sum(weighted[:, 0:2, :], axis=1)
            + jnp.where(first_half, g2, zero))
    win1 = (jnp.sum(weighted[:, 3:7, :], axis=1)
            + jnp.where(first_half, zero, g2)
            + jnp.where(first_half, g7, zero))
    win2 = (jnp.sum(weighted[:, 8:20, :], axis=1)
            + jnp.where(first_half, zero, g7))
    win3 = jnp.sum(weighted[:, 20:50, :], axis=1)

    parts = []
    for wsum, st, wn in zip((win0, win1, win2, win3), CUMSUM, WINDOWS):
        folded = jnp.dot(wsum, fold_ref[:], preferred_element_type=f32)
        denom = jnp.maximum(jnp.minimum(slen - st, wn), 1).astype(f32)
        parts.append(folded * (1.0 / denom))
    parts.append(q)
    out_ref[:] = jnp.concatenate(parts, axis=-1)


def kernel(query, sequence, sequence_length, W1, b1, W2, b2, Wl, bl, alpha):
    bf16 = jnp.bfloat16
    slen = sequence_length.astype(jnp.int32).reshape(B, 1)
    eye = jnp.eye(P, dtype=jnp.float32)
    w1cat = jnp.concatenate([jnp.kron(eye, W1[:D]),
                             jnp.kron(eye, W1[D:2 * D])],
                            axis=0).astype(bf16)
    w1c = jnp.tile(W1[2 * D:], (1, P)).astype(bf16)
    b1r = jnp.tile(b1.reshape(1, H1), (1, P))
    b2r = jnp.tile(b2.reshape(1, H2), (1, P))
    w2bd = jnp.kron(eye, W2).astype(bf16)
    wlmat = jnp.kron(eye, jnp.tile(Wl, (1, D))).astype(bf16)
    blr = bl.reshape(1, 1)
    alphar = alpha.reshape(1, 1)
    seqp = sequence.reshape(B, G, P * D)
    qtile = jnp.tile(query, (1, P)).astype(bf16)
    tpos = (P * jnp.arange(G, dtype=jnp.int32)[None, :, None]
            + (jnp.arange(P * D, dtype=jnp.int32) // D)[None, None, :])
    foldm = jnp.tile(jnp.eye(D, dtype=jnp.float32), (P, 1))

    grid = (B // BB,)
    full = lambda *s: pl.BlockSpec(s, lambda i: (0,) * len(s))
    out = pl.pallas_call(
        _din_block,
        grid=grid,
        in_specs=[
            pl.BlockSpec((BB, D), lambda i: (i, 0)),
            pl.BlockSpec((BB, P * D), lambda i: (i, 0)),
            pl.BlockSpec((BB, G, P * D), lambda i: (i, 0, 0)),
            pl.BlockSpec((BB, 1), lambda i: (i, 0)),
            full(2 * P * D, P * H1),
            full(D, P * H1),
            full(1, P * H1),
            full(P * H1, P * H2),
            full(1, P * H2),
            full(P * H2, P * D),
            full(1, 1),
            full(1, 1),
            full(1, G, P * D),
            full(P * D, D),
        ],
        out_specs=pl.BlockSpec((BB, 5 * D), lambda i: (i, 0)),
        out_shape=jax.ShapeDtypeStruct((B, 5 * D), jnp.float32),
        compiler_params=pltpu.CompilerParams(
            dimension_semantics=("parallel",),
        ),
    )(query, qtile, seqp, slen, w1cat, w1c, b1r, w2bd, b2r, wlmat, blr,
      alphar, tpos, foldm)
    return out
